# baseline (speedup 1.0000x reference)
import functools

import jax
import jax.numpy as jnp
from jax import lax
from jax.experimental import pallas as pl
from jax.experimental.pallas import tpu as pltpu
from jax.experimental.pallas import tpu_sc as plsc

_B = 512
_G = 28
_NB = 2
_CL = 80
_CH = _NB * 5 + _CL
_CHP = 96
_T = 16384
_BQ = _B // 128
_PW = _CHP // 2
_NGRAN = _G * _G * _BQ * _PW * 128 // 16

_NW = 32
_TPW = _T // _NW
_GCHUNK = 128
_NCHUNK = _TPW // _GCHUNK
_NSLOT = 12


def _ssqrt(v):
    a = jnp.abs(v) + 1e-6
    bits = plsc.bitcast(a, jnp.int32)
    bits = jnp.int32(0x5F3759DF) - lax.shift_right_arithmetic(bits, 1)
    y = plsc.bitcast(bits, jnp.float32)
    y = y * (1.5 - 0.5 * a * y * y)
    y = y * (1.5 - 0.5 * a * y * y)
    y = y * (1.5 - 0.5 * a * y * y)
    return jnp.sign(v) * (a * y)


_mesh = plsc.VectorSubcoreMesh(core_axis_name="c", subcore_axis_name="s")


@functools.partial(
    pl.kernel,
    mesh=_mesh,
    compiler_params=pltpu.CompilerParams(
        needs_layout_passes=False, use_tc_tiling_on_sc=False
    ),
    out_type=jax.ShapeDtypeStruct((_NW, 16), jnp.float32),
    scratch_types=[
        pltpu.VMEM((_TPW, 8), jnp.float32),
        pltpu.VMEM((_NCHUNK, _NSLOT, _GCHUNK), jnp.int32),
        pltpu.VMEM((_TPW,), jnp.int32),
        pltpu.VMEM((_TPW,), jnp.int32),
        pltpu.VMEM((_NSLOT, _TPW, 16), jnp.int32),
        pltpu.VMEM((16,), jnp.float32),
        pltpu.SemaphoreType.DMA,
    ],
)
def _sc_loss(gran_hbm, tgt_hbm, part_hbm, tgt_v, idx_v, lane_v, half_v,
             rows_v, acc_v, sem):
    wid = lax.axis_index("s") * 2 + lax.axis_index("c")
    base = wid * _TPW
    pltpu.sync_copy(tgt_hbm.at[pl.ds(base, _TPW)], tgt_v)

    lanes = lax.iota(jnp.int32, 16)

    def colv(c):
        return jnp.full((16,), c, jnp.int32)

    def tcol(rid, c):
        return plsc.load_gather(tgt_v, [rid, colv(c)])

    def idx_body(i, carry):
        rid = i * 16 + lanes
        bid = tcol(rid, 7).astype(jnp.int32)
        gx = tcol(rid, 4).astype(jnp.int32)
        gy = tcol(rid, 5).astype(jnp.int32)
        cls = tcol(rid, 6).astype(jnp.int32)
        cell = gx * _G + gy
        bhi = lax.shift_right_logical(bid, 4)
        lane_v[pl.ds(i * 16, 16)] = bid - lax.shift_left(bhi, 4)
        cslot = cls + 10
        chigh = (cslot >= _PW).astype(jnp.int32)
        half_v[pl.ds(i * 16, 16)] = chigh
        cword = cslot - chigh * _PW
        gbase = (
            (cell * _BQ + lax.shift_right_logical(bid, 7)) * (_PW * 8)
            + (bhi - lax.shift_left(lax.shift_right_logical(bid, 7), 3))
        )
        for s in range(10):
            idx_v[i // 8, s, pl.ds((i % 8) * 16, 16)] = gbase + s * 8
        idx_v[i // 8, 10, pl.ds((i % 8) * 16, 16)] = gbase + (90 - _PW) * 8
        idx_v[i // 8, 11, pl.ds((i % 8) * 16, 16)] = gbase + cword * 8
        return carry

    lax.fori_loop(0, _TPW // 16, idx_body, 0)

    copies = [
        pltpu.async_copy(
            gran_hbm.at[idx_v.at[j, s]],
            rows_v.at[s, pl.ds(j * _GCHUNK, _GCHUNK)],
            sem,
        )
        for j in range(_NCHUNK)
        for s in range(_NSLOT)
    ]
    for c in copies:
        c.wait()

    def body(i, acc):
        rid = i * 16 + lanes
        lanev = lane_v[pl.ds(i * 16, 16)]
        halfv = half_v[pl.ds(i * 16, 16)]
        hi_mask = jnp.full((16,), jnp.int32(-65536))

        def chan(s, half):
            v32 = plsc.load_gather(rows_v, [colv(s), rid, lanev])
            if half is None:
                bits = jnp.where(
                    halfv == 1,
                    jnp.bitwise_and(v32, hi_mask),
                    lax.shift_left(v32, 16),
                )
            elif half:
                bits = jnp.bitwise_and(v32, hi_mask)
            else:
                bits = lax.shift_left(v32, 16)
            return plsc.bitcast(bits, jnp.float32)

        xt = tcol(rid, 0)
        yt = tcol(rid, 1)
        wt = tcol(rid, 2)
        ht = tcol(rid, 3)

        tt = yt - 3.5 * ht
        bt = yt + 3.5 * ht
        lt = xt - 3.5 * wt
        rt = xt + 3.5 * wt
        at = wt * ht * 49.0

        ious = []
        boxes = []
        for nb in range(_NB):
            xg = chan(nb * 5 + 0, False)
            yg = chan(nb * 5 + 1, False)
            wg = chan(nb * 5 + 2, False)
            hg = chan(nb * 5 + 3, False)
            cg = chan(nb * 5 + 4, False)
            tg = yg - 3.5 * hg
            bg = yg + 3.5 * hg
            lg = xg - 3.5 * wg
            rg = xg + 3.5 * wg
            wi = jnp.maximum(jnp.minimum(rg, rt) - jnp.maximum(lg, lt), 0.0)
            hi = jnp.maximum(jnp.minimum(bg, bt) - jnp.maximum(tg, tt), 0.0)
            ai = wi * hi
            ag = wg * hg * 49.0
            tot = at + ag - ai
            safe = jnp.where(tot > 1e-6, tot, 1.0)
            ious.append(jnp.where(tot > 1e-6, ai / safe, 0.0))
            boxes.append((xg, yg, wg, hg, cg))

        sel = ious[1] > ious[0]
        xr, yr, wr, hr, cr = (
            jnp.where(sel, b1, b0) for b0, b1 in zip(boxes[0], boxes[1])
        )

        dx = xt - xr
        dy = yt - yr
        dw = _ssqrt(wt) - _ssqrt(wr)
        dh = _ssqrt(ht) - _ssqrt(hr)
        cm1 = cr - 1.0
        acc = acc + 5.0 * (dx * dx + dy * dy)
        acc = acc + 5.0 * (dw * dw + dh * dh)
        acc = acc + cm1 * cm1 - 0.5 * cr * cr
        acc = acc + chan(10, True)
        acc = acc + 1.0 - 2.0 * chan(11, None)
        return acc

    acc = lax.fori_loop(0, _TPW // 16, body, jnp.zeros((16,), jnp.float32))
    acc_v[...] = acc
    pltpu.sync_copy(acc_v, part_hbm.at[wid])


def _bf16_bits(x):
    b = lax.bitcast_convert_type(x, jnp.int32)
    r = b + 0x7FFF + jnp.bitwise_and(lax.shift_right_logical(b, 16), 1)
    return lax.shift_right_logical(r, 16)


def _tc_body(x_ref, img_ref, o_ref):
    step = pl.program_id(0) * pl.num_programs(1) + pl.program_id(1)

    @pl.when(step == 0)
    def _init():
        o_ref[...] = jnp.zeros((1, 1), jnp.float32)

    blk = x_ref[0]
    cls = blk[:, 10:_CH, :]
    u = jnp.sum(cls * cls, axis=1, keepdims=True)
    lo = blk[:, 0:_PW, :]
    hi = jnp.concatenate(
        [blk[:, _PW:_CH, :], u, jnp.zeros((_G, _CHP - _CH - 1, 128), jnp.float32)],
        axis=1,
    )
    img_ref[0, :, 0, :, :] = jnp.bitwise_or(
        _bf16_bits(lo), lax.shift_left(_bf16_bits(hi), 16)
    )
    c4 = blk[:, 4:5, :]
    c9 = blk[:, 9:10, :]
    noobj = jnp.sum(c4 * c4) + jnp.sum(c9 * c9)
    o_ref[...] = o_ref[...] + (0.5 * noobj).reshape(1, 1)


_tc_extract = pl.pallas_call(
    _tc_body,
    grid=(_G, _BQ),
    in_specs=[pl.BlockSpec((1, _G, _CH, 128), lambda i, q: (i, 0, 0, q))],
    out_specs=[
        pl.BlockSpec((1, _G, 1, _PW, 128), lambda i, q: (i, 0, q, 0, 0)),
        pl.BlockSpec((1, 1), lambda i, q: (0, 0)),
    ],
    out_shape=[
        jax.ShapeDtypeStruct((_G, _G, _BQ, _PW, 128), jnp.int32),
        jax.ShapeDtypeStruct((1, 1), jnp.float32),
    ],
)


def kernel(output, target):
    xt = jnp.transpose(output, (1, 2, 3, 0))
    img, noobj = _tc_extract(xt)
    part = _sc_loss(img.reshape(_NGRAN, 16), target)
    return jnp.sum(part) + noobj[0, 0]


# speedup vs baseline: 1.2817x; 1.2817x over previous
import functools

import jax
import jax.numpy as jnp
from jax import lax
from jax.experimental import pallas as pl
from jax.experimental.pallas import tpu as pltpu
from jax.experimental.pallas import tpu_sc as plsc

_B = 512
_G = 28
_NB = 2
_CL = 80
_CH = _NB * 5 + _CL
_CHP = 96
_T = 16384
_BQ = _B // 128
_PW = _CHP // 2
_NGRAN = _G * _G * _BQ * _PW * 128 // 16

_NW = 32
_TPW = _T // _NW
_GCHUNK = 128
_NCHUNK = _TPW // _GCHUNK
_NSLOT = 12


def _ssqrt(v):
    a = jnp.abs(v) + 1e-6
    bits = plsc.bitcast(a, jnp.int32)
    bits = jnp.int32(0x5F3759DF) - lax.shift_right_arithmetic(bits, 1)
    y = plsc.bitcast(bits, jnp.float32)
    y = y * (1.5 - 0.5 * a * y * y)
    y = y * (1.5 - 0.5 * a * y * y)
    y = y * (1.5 - 0.5 * a * y * y)
    return jnp.sign(v) * (a * y)


_mesh = plsc.VectorSubcoreMesh(core_axis_name="c", subcore_axis_name="s")


@functools.partial(
    pl.kernel,
    mesh=_mesh,
    compiler_params=pltpu.CompilerParams(
        needs_layout_passes=False, use_tc_tiling_on_sc=False
    ),
    out_type=jax.ShapeDtypeStruct((_NW, 16), jnp.float32),
    scratch_types=[
        pltpu.VMEM((_TPW, 8), jnp.float32),
        pltpu.VMEM((_NCHUNK, _NSLOT, _GCHUNK), jnp.int32),
        pltpu.VMEM((_TPW,), jnp.int32),
        pltpu.VMEM((_TPW,), jnp.int32),
        pltpu.VMEM((_NSLOT, _TPW, 16), jnp.int32),
        pltpu.VMEM((16,), jnp.float32),
        pltpu.SemaphoreType.DMA,
    ],
)
def _sc_loss(gran_hbm, tgt_hbm, part_hbm, tgt_v, idx_v, lane_v, half_v,
             rows_v, acc_v, sem):
    wid = lax.axis_index("s") * 2 + lax.axis_index("c")
    base = wid * _TPW
    pltpu.sync_copy(tgt_hbm.at[pl.ds(base, _TPW)], tgt_v)

    lanes = lax.iota(jnp.int32, 16)

    def colv(c):
        return jnp.full((16,), c, jnp.int32)

    def tcol(rid, c):
        return plsc.load_gather(tgt_v, [rid, colv(c)])

    def idx_body(i, carry):
        rid = i * 16 + lanes
        bid = tcol(rid, 7).astype(jnp.int32)
        gx = tcol(rid, 4).astype(jnp.int32)
        gy = tcol(rid, 5).astype(jnp.int32)
        cls = tcol(rid, 6).astype(jnp.int32)
        cell = gx * _G + gy
        bhi = lax.shift_right_logical(bid, 4)
        lane_v[pl.ds(i * 16, 16)] = bid - lax.shift_left(bhi, 4)
        cslot = cls + 10
        chigh = (cslot >= _PW).astype(jnp.int32)
        half_v[pl.ds(i * 16, 16)] = chigh
        cword = cslot - chigh * _PW
        gbase = (
            (cell * _BQ + lax.shift_right_logical(bid, 7)) * (_PW * 8)
            + (bhi - lax.shift_left(lax.shift_right_logical(bid, 7), 3))
        )
        for s in range(10):
            idx_v[i // 8, s, pl.ds((i % 8) * 16, 16)] = gbase + s * 8
        idx_v[i // 8, 10, pl.ds((i % 8) * 16, 16)] = gbase + (90 - _PW) * 8
        idx_v[i // 8, 11, pl.ds((i % 8) * 16, 16)] = gbase + cword * 8
        return carry

    lax.fori_loop(0, _TPW // 16, idx_body, 0)

    copies = [
        pltpu.async_copy(
            gran_hbm.at[idx_v.at[j, s]],
            rows_v.at[s, pl.ds(j * _GCHUNK, _GCHUNK)],
            sem,
        )
        for j in range(_NCHUNK)
        for s in range(_NSLOT)
    ]
    for c in copies:
        c.wait()

    def body(i, acc):
        rid = i * 16 + lanes
        lanev = lane_v[pl.ds(i * 16, 16)]
        halfv = half_v[pl.ds(i * 16, 16)]
        hi_mask = jnp.full((16,), jnp.int32(-65536))

        def chan(s, half):
            v32 = plsc.load_gather(rows_v, [colv(s), rid, lanev])
            if half is None:
                bits = jnp.where(
                    halfv == 1,
                    jnp.bitwise_and(v32, hi_mask),
                    lax.shift_left(v32, 16),
                )
            elif half:
                bits = jnp.bitwise_and(v32, hi_mask)
            else:
                bits = lax.shift_left(v32, 16)
            return plsc.bitcast(bits, jnp.float32)

        xt = tcol(rid, 0)
        yt = tcol(rid, 1)
        wt = tcol(rid, 2)
        ht = tcol(rid, 3)

        tt = yt - 3.5 * ht
        bt = yt + 3.5 * ht
        lt = xt - 3.5 * wt
        rt = xt + 3.5 * wt
        at = wt * ht * 49.0

        ious = []
        boxes = []
        for nb in range(_NB):
            xg = chan(nb * 5 + 0, False)
            yg = chan(nb * 5 + 1, False)
            wg = chan(nb * 5 + 2, False)
            hg = chan(nb * 5 + 3, False)
            cg = chan(nb * 5 + 4, False)
            tg = yg - 3.5 * hg
            bg = yg + 3.5 * hg
            lg = xg - 3.5 * wg
            rg = xg + 3.5 * wg
            wi = jnp.maximum(jnp.minimum(rg, rt) - jnp.maximum(lg, lt), 0.0)
            hi = jnp.maximum(jnp.minimum(bg, bt) - jnp.maximum(tg, tt), 0.0)
            ai = wi * hi
            ag = wg * hg * 49.0
            tot = at + ag - ai
            safe = jnp.where(tot > 1e-6, tot, 1.0)
            ious.append(jnp.where(tot > 1e-6, ai / safe, 0.0))
            boxes.append((xg, yg, wg, hg, cg))

        sel = ious[1] > ious[0]
        xr, yr, wr, hr, cr = (
            jnp.where(sel, b1, b0) for b0, b1 in zip(boxes[0], boxes[1])
        )

        dx = xt - xr
        dy = yt - yr
        dw = _ssqrt(wt) - _ssqrt(wr)
        dh = _ssqrt(ht) - _ssqrt(hr)
        cm1 = cr - 1.0
        acc = acc + 5.0 * (dx * dx + dy * dy)
        acc = acc + 5.0 * (dw * dw + dh * dh)
        acc = acc + cm1 * cm1 - 0.5 * cr * cr
        acc = acc + chan(10, True)
        acc = acc + 1.0 - 2.0 * chan(11, None)
        return acc

    acc = lax.fori_loop(0, _TPW // 16, body, jnp.zeros((16,), jnp.float32))
    acc_v[...] = acc
    pltpu.sync_copy(acc_v, part_hbm.at[wid])


def _bf16_bits(x):
    b = lax.bitcast_convert_type(x, jnp.int32)
    r = b + 0x7FFF + jnp.bitwise_and(lax.shift_right_logical(b, 16), 1)
    return lax.shift_right_logical(r, 16)


def _tc_body(x_ref, img_ref, o_ref):
    step = pl.program_id(0)

    @pl.when(step == 0)
    def _init():
        o_ref[...] = jnp.zeros((1, 1), jnp.float32)

    blk = x_ref[0]
    cls = blk[:, 10:_CH, :]
    u = jnp.sum(cls * cls, axis=1, keepdims=True)
    lo = blk[:, 0:_PW, :]
    hi = jnp.concatenate(
        [blk[:, _PW:_CH, :], u, jnp.zeros((_G, _CHP - _CH - 1, _B), jnp.float32)],
        axis=1,
    )
    packed = jnp.bitwise_or(
        _bf16_bits(lo), lax.shift_left(_bf16_bits(hi), 16)
    )
    for q in range(_BQ):
        img_ref[0, :, q, :, :] = packed[:, :, q * 128:(q + 1) * 128]
    c4 = blk[:, 4:5, :]
    c9 = blk[:, 9:10, :]
    noobj = jnp.sum(c4 * c4) + jnp.sum(c9 * c9)
    o_ref[...] = o_ref[...] + (0.5 * noobj).reshape(1, 1)


_tc_extract = pl.pallas_call(
    _tc_body,
    grid=(_G,),
    in_specs=[pl.BlockSpec((1, _G, _CH, _B), lambda i: (i, 0, 0, 0))],
    out_specs=[
        pl.BlockSpec((1, _G, _BQ, _PW, 128), lambda i: (i, 0, 0, 0, 0)),
        pl.BlockSpec((1, 1), lambda i: (0, 0)),
    ],
    out_shape=[
        jax.ShapeDtypeStruct((_G, _G, _BQ, _PW, 128), jnp.int32),
        jax.ShapeDtypeStruct((1, 1), jnp.float32),
    ],
)


def kernel(output, target):
    xt = jnp.transpose(output, (1, 2, 3, 0))
    img, noobj = _tc_extract(xt)
    part = _sc_loss(img.reshape(_NGRAN, 16), target)
    return jnp.sum(part) + noobj[0, 0]
